# baseline (speedup 1.0000x reference)
import functools

import numpy as np
import jax
import jax.numpy as jnp
from jax import lax
from jax.experimental import pallas as pl
from jax.experimental.pallas import tpu as pltpu
from jax.experimental.pallas import tpu_sc as plsc

N = 10000
E = 320000
D_IN = 128
D_HID = 256
MASK_RATE = 0.3
REPLACE_RATE = 0.1
ALPHA = 2.0

NP_PAD = 10240
ROW_BLK = 512
COL_BLK = 2048
PAD_CONST = 0.25 * (NP_PAD * NP_PAD - N * N)


def _mask_constants():
    try:
        with jax.ensure_compile_time_eval():
            return _mask_constants_impl()
    except Exception:
        return _mask_constants_impl()


def _mask_constants_impl():
    k = jax.random.key(1)
    k1, k2, k3 = jax.random.split(k, 3)
    perm = jax.random.permutation(k1, N)
    num_mask = int(MASK_RATE * N)
    mask_nodes = perm[:num_mask].astype(jnp.int32)
    num_noise = int(REPLACE_RATE * num_mask)
    perm_mask = jax.random.permutation(k2, num_mask)
    token_nodes = mask_nodes[perm_mask[: int((1.0 - REPLACE_RATE) * num_mask)]]
    noise_nodes = mask_nodes[perm_mask[num_mask - num_noise:]]
    noise_chosen = jax.random.permutation(k3, N)[:num_noise].astype(jnp.int32)
    gather_idx = jnp.arange(N, dtype=jnp.int32).at[noise_nodes].set(noise_chosen)
    token_flag = jnp.zeros((N, 1), jnp.float32).at[token_nodes].set(1.0)
    mask_keep = jnp.ones((N, 1), jnp.float32).at[mask_nodes].set(0.0)
    return mask_nodes, gather_idx, token_flag, mask_keep


N_PADROW = 10240
EDGE_CHUNK = 128
CHUNKS_PER_SUB = 158
EDGES_PAD = 16 * EDGE_CHUNK * CHUNKS_PER_SUB
CHUNKS_PER_CS = 80
EDGES_PAD_ES = 32 * EDGE_CHUNK * CHUNKS_PER_CS
ROWS_PER_SUB = N_PADROW // 16


@functools.lru_cache(maxsize=None)
def _make_prop(d2, edge_split):
    mesh = plsc.VectorSubcoreMesh(core_axis_name="c", subcore_axis_name="s")
    nch = CHUNKS_PER_CS if edge_split else CHUNKS_PER_SUB

    @functools.partial(
        pl.kernel,
        out_type=jax.ShapeDtypeStruct((2, N_PADROW, d2), jnp.float32),
        mesh=mesh,
        scratch_types=[
            [pltpu.VMEM((EDGE_CHUNK,), jnp.int32)] * 2,
            [pltpu.VMEM((EDGE_CHUNK,), jnp.int32)] * 2,
            [pltpu.VMEM((EDGE_CHUNK, d2), jnp.float32)] * 2,
            pltpu.VMEM_SHARED((N_PADROW, d2), jnp.float32),
            [pltpu.SemaphoreType.DMA] * 2,
        ],
    )
    def prop_kernel(hcat, src2, dst2, out, sidx, didx, rows, acc, sems):
        c = lax.axis_index("c")
        s = lax.axis_index("s")
        r0 = s * ROWS_PER_SUB
        pltpu.sync_copy(hcat.at[pl.ds(c * N_PADROW + r0, ROWS_PER_SUB), :],
                        acc.at[pl.ds(r0, ROWS_PER_SUB), :])
        plsc.subcore_barrier()

        w = (c * 16 + s) if edge_split else s

        def load_and_fire(b, g):
            base = (w * nch + g) * EDGE_CHUNK
            pltpu.sync_copy(src2.at[c, pl.ds(base, EDGE_CHUNK)], sidx[b])
            pltpu.sync_copy(dst2.at[c, pl.ds(base, EDGE_CHUNK)], didx[b])
            pltpu.async_copy(hcat.at[sidx[b]], rows[b], sems[b])

        def drain_and_scatter(b):
            pltpu.make_async_copy(hcat.at[sidx[b]], rows[b], sems[b]).wait()
            pltpu.sync_copy(rows[b], acc.at[didx[b]], add=True)

        for b in (0, 1):
            load_and_fire(b, b)

        @pl.loop(0, nch - 2, step=2)
        def _(g):
            for b in (0, 1):
                drain_and_scatter(b)
                load_and_fire(b, g + b + 2)

        for b in (0, 1):
            drain_and_scatter(b)

        plsc.subcore_barrier()
        pltpu.sync_copy(acc.at[pl.ds(r0, ROWS_PER_SUB), :],
                        out.at[c, pl.ds(r0, ROWS_PER_SUB), :])

    return prop_kernel


def _sc_prop(h, src2, dst2, d2):
    hs = jnp.transpose(h.reshape(N, 2, d2), (1, 0, 2))
    hcat = jnp.zeros((2, N_PADROW, d2), h.dtype).at[:, :N, :].set(hs)
    hcat = hcat.reshape(2 * N_PADROW, d2)
    out = _make_prop(d2, False)(hcat, src2, dst2)
    return jnp.transpose(out[:, :N, :], (1, 0, 2)).reshape(N, 2 * d2)


def _sc_prop_es(h, src2_es, dst2_es):
    hcat2 = jnp.zeros((2 * N_PADROW, 128), h.dtype).at[:N].set(h)
    out = _make_prop(128, True)(hcat2, src2_es, dst2_es)
    return out[0, :N, :] + out[1, :N, :]


def _sq_loss_kernel(zr_ref, zc_ref, out_ref):
    j = pl.program_id(1)
    t = lax.dot_general(zr_ref[...], zc_ref[...],
                        (((1,), (1,)), ((), ())),
                        preferred_element_type=jnp.float32)
    s = jax.nn.sigmoid(t)
    part = jnp.sum(s * s)
    lane0 = lax.broadcasted_iota(jnp.int32, (1, 8, 128), 2) == 0

    @pl.when(j == 0)
    def _():
        out_ref[...] = jnp.zeros_like(out_ref)

    out_ref[...] += jnp.where(lane0, part, 0.0)


def _sq_loss(zp):
    ni = NP_PAD // ROW_BLK
    nj = NP_PAD // COL_BLK
    out = pl.pallas_call(
        _sq_loss_kernel,
        grid=(ni, nj),
        in_specs=[
            pl.BlockSpec((ROW_BLK, D_IN), lambda i, j: (i, 0)),
            pl.BlockSpec((COL_BLK, D_IN), lambda i, j: (j, 0)),
        ],
        out_specs=pl.BlockSpec((1, 8, 128), lambda i, j: (i, 0, 0)),
        out_shape=jax.ShapeDtypeStruct((ni, 8, 128), jnp.float32),
    )(zp, zp)
    return jnp.sum(out[:, 0, 0])


def kernel(x, edge_index, W_enc1, W_enc2, W_e2d, W_attr_dec, W_struct_dec, enc_mask_token):
    mask_nodes, gather_idx, token_flag, mask_keep = _mask_constants()
    src = edge_index[0].astype(jnp.int32)
    dst = edge_index[1].astype(jnp.int32)

    use_x = jnp.where(token_flag > 0, enc_mask_token[0][None, :],
                      jnp.take(x, gather_idx, axis=0))

    deg = jnp.zeros((N,), jnp.float32).at[dst].add(1.0) + 1.0
    dis = lax.rsqrt(deg)[:, None]

    npad = EDGES_PAD - E
    srcp = jnp.concatenate([src, jnp.full((npad,), N, jnp.int32)])
    src2 = jnp.stack([srcp, srcp + N_PADROW])
    dstp = jnp.concatenate([dst, jnp.full((npad,), N, jnp.int32)])
    dst2 = jnp.stack([dstp, dstp])
    npad_es = EDGES_PAD_ES - E
    srcp_es = jnp.concatenate([src, jnp.full((npad_es,), N, jnp.int32)])
    src2_es = jnp.stack([srcp_es, srcp_es])
    dstp_es = jnp.concatenate([dst, jnp.full((npad_es,), N, jnp.int32)])
    dst2_es = jnp.stack([dstp_es, dstp_es])

    def prop(h):
        if h.shape[1] == 128:
            return _sc_prop_es(h, src2_es, dst2_es)
        return _sc_prop(h, src2, dst2, h.shape[1] // 2)

    t0 = use_x * dis
    h1 = jnp.maximum(prop(t0) * dis @ W_enc1, 0.0)
    t1 = h1 * dis
    h2 = jnp.maximum(prop(t1) * dis @ W_enc2, 0.0)
    rep = h2 @ W_e2d
    t2 = rep * (dis * mask_keep)
    q = prop(t2) * dis
    z = q @ W_struct_dec

    mask_idx = mask_nodes
    pred = jnp.take(q, mask_idx, axis=0) @ W_attr_dec
    tgt = jnp.take(x, mask_idx, axis=0)
    pn = pred / (jnp.linalg.norm(pred, axis=-1, keepdims=True) + 1e-8)
    tn = tgt / (jnp.linalg.norm(tgt, axis=-1, keepdims=True) + 1e-8)
    attr_loss = jnp.mean((1.0 - jnp.sum(pn * tn, axis=-1)) ** ALPHA)

    zp = jnp.zeros((NP_PAD, D_IN), jnp.float32).at[:N].set(z)
    ssum = _sq_loss(zp) - PAD_CONST

    key = src * N + dst
    ks = jnp.sort(key)
    first = jnp.concatenate([jnp.ones((1,), jnp.bool_), ks[1:] != ks[:-1]])
    es, ed = ks // N, ks % N
    dots = jnp.sum(jnp.take(z, es, axis=0) * jnp.take(z, ed, axis=0), axis=-1)
    corr = jnp.sum(jnp.where(first, 1.0 - 2.0 * jax.nn.sigmoid(dots), 0.0))

    struct_loss = (ssum + corr) / (N * N)
    return attr_loss + struct_loss


# speedup vs baseline: 1.0318x; 1.0318x over previous
import functools

import numpy as np
import jax
import jax.numpy as jnp
from jax import lax
from jax.experimental import pallas as pl
from jax.experimental.pallas import tpu as pltpu
from jax.experimental.pallas import tpu_sc as plsc

N = 10000
E = 320000
D_IN = 128
D_HID = 256
MASK_RATE = 0.3
REPLACE_RATE = 0.1
ALPHA = 2.0

NP_PAD = 10240
ROW_BLK = 512
COL_BLK = 2048
PAD_CONST = 0.25 * (NP_PAD * NP_PAD - N * N)


def _mask_constants():
    try:
        with jax.ensure_compile_time_eval():
            return _mask_constants_impl()
    except Exception:
        return _mask_constants_impl()


def _mask_constants_impl():
    k = jax.random.key(1)
    k1, k2, k3 = jax.random.split(k, 3)
    perm = jax.random.permutation(k1, N)
    num_mask = int(MASK_RATE * N)
    mask_nodes = perm[:num_mask].astype(jnp.int32)
    num_noise = int(REPLACE_RATE * num_mask)
    perm_mask = jax.random.permutation(k2, num_mask)
    token_nodes = mask_nodes[perm_mask[: int((1.0 - REPLACE_RATE) * num_mask)]]
    noise_nodes = mask_nodes[perm_mask[num_mask - num_noise:]]
    noise_chosen = jax.random.permutation(k3, N)[:num_noise].astype(jnp.int32)
    gather_idx = jnp.arange(N, dtype=jnp.int32).at[noise_nodes].set(noise_chosen)
    token_flag = jnp.zeros((N, 1), jnp.float32).at[token_nodes].set(1.0)
    mask_keep = jnp.ones((N, 1), jnp.float32).at[mask_nodes].set(0.0)
    return mask_nodes, gather_idx, token_flag, mask_keep


N_PADROW = 10240
EDGE_CHUNK = 128
CHUNKS_PER_SUB = 158
EDGES_PAD = 16 * EDGE_CHUNK * CHUNKS_PER_SUB
CHUNKS_PER_CS = 80
EDGES_PAD_ES = 32 * EDGE_CHUNK * CHUNKS_PER_CS
ROWS_PER_SUB = N_PADROW // 16


@functools.lru_cache(maxsize=None)
def _make_prop(d2, edge_split):
    mesh = plsc.VectorSubcoreMesh(core_axis_name="c", subcore_axis_name="s")
    nch = CHUNKS_PER_CS if edge_split else CHUNKS_PER_SUB

    @functools.partial(
        pl.kernel,
        out_type=jax.ShapeDtypeStruct((2, N_PADROW, d2), jnp.float32),
        mesh=mesh,
        scratch_types=[
            [pltpu.VMEM((EDGE_CHUNK,), jnp.int32)] * 2,
            [pltpu.VMEM((EDGE_CHUNK,), jnp.int32)] * 2,
            [pltpu.VMEM((EDGE_CHUNK, d2), jnp.float32)] * 2,
            pltpu.VMEM_SHARED((N_PADROW, d2), jnp.float32),
            [pltpu.SemaphoreType.DMA] * 2,
        ],
    )
    def prop_kernel(hcat, src2, dst2, out, sidx, didx, rows, acc, sems):
        c = lax.axis_index("c")
        s = lax.axis_index("s")
        r0 = s * ROWS_PER_SUB
        pltpu.sync_copy(hcat.at[pl.ds(c * N_PADROW + r0, ROWS_PER_SUB), :],
                        acc.at[pl.ds(r0, ROWS_PER_SUB), :])
        plsc.subcore_barrier()

        w = (c * 16 + s) if edge_split else s

        def load_and_fire(b, g):
            base = (w * nch + g) * EDGE_CHUNK
            pltpu.sync_copy(src2.at[c, pl.ds(base, EDGE_CHUNK)], sidx[b])
            pltpu.sync_copy(dst2.at[c, pl.ds(base, EDGE_CHUNK)], didx[b])
            pltpu.async_copy(hcat.at[sidx[b]], rows[b], sems[b])

        def drain_and_scatter(b):
            pltpu.make_async_copy(hcat.at[sidx[b]], rows[b], sems[b]).wait()
            pltpu.sync_copy(rows[b], acc.at[didx[b]], add=True)

        for b in (0, 1):
            load_and_fire(b, b)

        @pl.loop(0, nch - 2, step=2)
        def _(g):
            for b in (0, 1):
                drain_and_scatter(b)
                load_and_fire(b, g + b + 2)

        for b in (0, 1):
            drain_and_scatter(b)

        plsc.subcore_barrier()
        pltpu.sync_copy(acc.at[pl.ds(r0, ROWS_PER_SUB), :],
                        out.at[c, pl.ds(r0, ROWS_PER_SUB), :])

    return prop_kernel


def _sc_prop(h, src2, dst2, d2):
    hs = jnp.transpose(h.reshape(N, 2, d2), (1, 0, 2))
    hcat = jnp.zeros((2, N_PADROW, d2), h.dtype).at[:, :N, :].set(hs)
    hcat = hcat.reshape(2 * N_PADROW, d2)
    out = _make_prop(d2, False)(hcat, src2, dst2)
    return jnp.transpose(out[:, :N, :], (1, 0, 2)).reshape(N, 2 * d2)


def _sc_prop_es(h, src2_es, dst2_es):
    hcat2 = jnp.zeros((2 * N_PADROW, 128), h.dtype).at[:N].set(h)
    out = _make_prop(128, True)(hcat2, src2_es, dst2_es)
    return out[0, :N, :] + out[1, :N, :]


@functools.lru_cache(maxsize=None)
def _make_edge_gather():
    mesh = plsc.VectorSubcoreMesh(core_axis_name="c", subcore_axis_name="s")
    nch = CHUNKS_PER_CS

    @functools.partial(
        pl.kernel,
        out_type=(jax.ShapeDtypeStruct((EDGES_PAD_ES, 128), jnp.float32),
                  jax.ShapeDtypeStruct((EDGES_PAD_ES, 128), jnp.float32)),
        mesh=mesh,
        scratch_types=[
            [pltpu.VMEM((EDGE_CHUNK,), jnp.int32)] * 2,
            [pltpu.VMEM((EDGE_CHUNK,), jnp.int32)] * 2,
            [pltpu.VMEM((EDGE_CHUNK, 128), jnp.float32)] * 2,
            [pltpu.VMEM((EDGE_CHUNK, 128), jnp.float32)] * 2,
            [pltpu.SemaphoreType.DMA] * 2,
            [pltpu.SemaphoreType.DMA] * 2,
        ],
    )
    def gather_kernel(zcat, esp, edp, zrg, zdg, sidx, didx, zr, zd, gs, gd):
        c = lax.axis_index("c")
        s = lax.axis_index("s")
        w = c * 16 + s

        def fire(b, g):
            base = (w * nch + g) * EDGE_CHUNK
            pltpu.sync_copy(esp.at[pl.ds(base, EDGE_CHUNK)], sidx[b])
            pltpu.sync_copy(edp.at[pl.ds(base, EDGE_CHUNK)], didx[b])
            pltpu.async_copy(zcat.at[sidx[b]], zr[b], gs[b])
            pltpu.async_copy(zcat.at[didx[b]], zd[b], gd[b])

        def drain_store(b, g):
            base = (w * nch + g) * EDGE_CHUNK
            pltpu.make_async_copy(zcat.at[sidx[b]], zr[b], gs[b]).wait()
            pltpu.make_async_copy(zcat.at[didx[b]], zd[b], gd[b]).wait()
            pltpu.sync_copy(zr[b], zrg.at[pl.ds(base, EDGE_CHUNK), :])
            pltpu.sync_copy(zd[b], zdg.at[pl.ds(base, EDGE_CHUNK), :])

        for b in (0, 1):
            fire(b, b)

        @pl.loop(0, nch - 2, step=2)
        def _(g):
            for b in (0, 1):
                drain_store(b, g + b)
                fire(b, g + b + 2)

        for b in (0, 1):
            drain_store(b, nch - 2 + b)

    return gather_kernel


BLKE = 2048


def _edge_dot_kernel(zr_ref, zd_ref, m_ref, out_ref):
    i = pl.program_id(0)
    t = jnp.sum(zr_ref[...] * zd_ref[...], axis=1)
    sv = jax.nn.sigmoid(t)
    p = jnp.sum(m_ref[...] * sv.reshape(BLKE // 128, 128))
    lane0 = lax.broadcasted_iota(jnp.int32, (1, 8, 128), 2) == 0

    @pl.when(i == 0)
    def _():
        out_ref[...] = jnp.zeros_like(out_ref)

    out_ref[...] += jnp.where(lane0, p, 0.0)


def _edge_dot_sum(zrg, zdg, mp):
    nb = EDGES_PAD_ES // BLKE
    m2 = mp.reshape(EDGES_PAD_ES // 128, 128)
    out = pl.pallas_call(
        _edge_dot_kernel,
        grid=(nb,),
        in_specs=[
            pl.BlockSpec((BLKE, 128), lambda i: (i, 0)),
            pl.BlockSpec((BLKE, 128), lambda i: (i, 0)),
            pl.BlockSpec((BLKE // 128, 128), lambda i: (i, 0)),
        ],
        out_specs=pl.BlockSpec((1, 8, 128), lambda i: (0, 0, 0)),
        out_shape=jax.ShapeDtypeStruct((1, 8, 128), jnp.float32),
    )(zrg, zdg, m2)
    return out[0, 0, 0]


def _sq_loss_kernel(zr_ref, zc_ref, out_ref):
    j = pl.program_id(1)
    t = lax.dot_general(zr_ref[...], zc_ref[...],
                        (((1,), (1,)), ((), ())),
                        preferred_element_type=jnp.float32)
    s = jax.nn.sigmoid(t)
    part = jnp.sum(s * s)
    lane0 = lax.broadcasted_iota(jnp.int32, (1, 8, 128), 2) == 0

    @pl.when(j == 0)
    def _():
        out_ref[...] = jnp.zeros_like(out_ref)

    out_ref[...] += jnp.where(lane0, part, 0.0)


def _sq_loss(zp):
    ni = NP_PAD // ROW_BLK
    nj = NP_PAD // COL_BLK
    out = pl.pallas_call(
        _sq_loss_kernel,
        grid=(ni, nj),
        in_specs=[
            pl.BlockSpec((ROW_BLK, D_IN), lambda i, j: (i, 0)),
            pl.BlockSpec((COL_BLK, D_IN), lambda i, j: (j, 0)),
        ],
        out_specs=pl.BlockSpec((1, 8, 128), lambda i, j: (i, 0, 0)),
        out_shape=jax.ShapeDtypeStruct((ni, 8, 128), jnp.float32),
    )(zp, zp)
    return jnp.sum(out[:, 0, 0])


def kernel(x, edge_index, W_enc1, W_enc2, W_e2d, W_attr_dec, W_struct_dec, enc_mask_token):
    mask_nodes, gather_idx, token_flag, mask_keep = _mask_constants()
    src = edge_index[0].astype(jnp.int32)
    dst = edge_index[1].astype(jnp.int32)

    use_x = jnp.where(token_flag > 0, enc_mask_token[0][None, :],
                      jnp.take(x, gather_idx, axis=0))

    deg = jnp.zeros((N,), jnp.float32).at[dst].add(1.0) + 1.0
    dis = lax.rsqrt(deg)[:, None]

    npad = EDGES_PAD - E
    srcp = jnp.concatenate([src, jnp.full((npad,), N, jnp.int32)])
    src2 = jnp.stack([srcp, srcp + N_PADROW])
    dstp = jnp.concatenate([dst, jnp.full((npad,), N, jnp.int32)])
    dst2 = jnp.stack([dstp, dstp])
    npad_es = EDGES_PAD_ES - E
    srcp_es = jnp.concatenate([src, jnp.full((npad_es,), N, jnp.int32)])
    src2_es = jnp.stack([srcp_es, srcp_es])
    dstp_es = jnp.concatenate([dst, jnp.full((npad_es,), N, jnp.int32)])
    dst2_es = jnp.stack([dstp_es, dstp_es])

    def prop(h):
        if h.shape[1] == 128:
            return _sc_prop_es(h, src2_es, dst2_es)
        return _sc_prop(h, src2, dst2, h.shape[1] // 2)

    t0 = use_x * dis
    h1 = jnp.maximum(prop(t0) * dis @ W_enc1, 0.0)
    t1 = h1 * dis
    h2 = jnp.maximum(prop(t1) * dis @ W_enc2, 0.0)
    rep = h2 @ W_e2d
    t2 = rep * (dis * mask_keep)
    q = prop(t2) * dis
    z = q @ W_struct_dec

    mask_idx = mask_nodes
    pred = jnp.take(q, mask_idx, axis=0) @ W_attr_dec
    tgt = jnp.take(x, mask_idx, axis=0)
    pn = pred / (jnp.linalg.norm(pred, axis=-1, keepdims=True) + 1e-8)
    tn = tgt / (jnp.linalg.norm(tgt, axis=-1, keepdims=True) + 1e-8)
    attr_loss = jnp.mean((1.0 - jnp.sum(pn * tn, axis=-1)) ** ALPHA)

    zp = jnp.zeros((NP_PAD, D_IN), jnp.float32).at[:N].set(z)
    ssum = _sq_loss(zp) - PAD_CONST

    key = src * N + dst
    ks = jnp.sort(key)
    first = jnp.concatenate([jnp.ones((1,), jnp.bool_), ks[1:] != ks[:-1]])
    m = first.astype(jnp.float32)
    es, ed = ks // N, ks % N
    npad_c = EDGES_PAD_ES - E
    esp = jnp.concatenate([es, jnp.full((npad_c,), N, jnp.int32)])
    edp = jnp.concatenate([ed, jnp.full((npad_c,), N, jnp.int32)])
    mp = jnp.concatenate([m, jnp.zeros((npad_c,), jnp.float32)])
    msum = jnp.sum(m)
    zrg, zdg = _make_edge_gather()(zp, esp, edp)
    corr = msum - 2.0 * _edge_dot_sum(zrg, zdg, mp)

    struct_loss = (ssum + corr) / (N * N)
    return attr_loss + struct_loss
